# Initial kernel scaffold; baseline (speedup 1.0000x reference)
import jax
import jax.numpy as jnp
from jax.experimental import pallas as pl


def kernel(ctrs, feats, pre_u, pre_v, suc_u, suc_v, idcs, in_W1, in_b1, in_W2, in_b2, in_g, in_be, seg_W1, seg_b1, seg_W2, seg_b2, seg_g, seg_be, ctr_W, pre_W, suc_W, norm_g, norm_b, ctr2_W, ctr2_b, ctr2_g, ctr2_be):
    raise NotImplementedError("write your pallas kernel here")


# speedup vs baseline: 3.3612x; 3.3612x over previous
import functools

import jax
import jax.numpy as jnp
from jax import lax
from jax.experimental import pallas as pl
from jax.experimental.pallas import tpu as pltpu
from jax.experimental.pallas import tpu_sc as plsc

N = 10000
S = 6
D = 256
NP = 10240
EP = 10240
NSETS = 2 * S
TILES = 16
ECHUNK = 128
QCH = EP // TILES // ECHUNK
NCHUNK = NSETS * QCH
NBUF = 3
CH = D // 2
HALF = NP // 2
TRROWS = 128
AROWS = HALF + TRROWS
ZPT = AROWS // TILES
WPT = HALF // TILES
ROWS = 640


def _gn_rows(x, g, b):
    m = jnp.mean(x, axis=-1, keepdims=True)
    v = jnp.mean((x - m) ** 2, axis=-1, keepdims=True)
    return (x - m) / jnp.sqrt(v + 1e-5) * g + b


def _write_y(feat, pre_w, suc_w, y_out):
    for k in range(S):
        z = jnp.dot(feat, pre_w[k], preferred_element_type=jnp.float32)
        y_out[0, k] = z[:, :CH]
        y_out[1, k] = z[:, CH:]
        z = jnp.dot(feat, suc_w[k], preferred_element_type=jnp.float32)
        y_out[0, S + k] = z[:, :CH]
        y_out[1, S + k] = z[:, CH:]


def _tc_first_body(ctrs_ref, feats_ref, in_w1, in_b1, in_w2, in_b2, in_g, in_be,
                   seg_w1, seg_b1, seg_w2, seg_b2, seg_g, seg_be,
                   pre_w, suc_w, feat_out, y_out):
    ctrs = ctrs_ref[...]
    feats = feats_ref[...]
    w1 = in_w1[...]
    h = jnp.maximum(ctrs[:, 0:1] * w1[0:1, :] + ctrs[:, 1:2] * w1[1:2, :] + in_b1[...], 0.0)
    h = _gn_rows(jnp.dot(h, in_w2[...], preferred_element_type=jnp.float32) + in_b2[...],
                 in_g[...], in_be[...])
    w1 = seg_w1[...]
    sg = jnp.maximum(feats[:, 0:1] * w1[0:1, :] + feats[:, 1:2] * w1[1:2, :] + seg_b1[...], 0.0)
    sg = _gn_rows(jnp.dot(sg, seg_w2[...], preferred_element_type=jnp.float32) + seg_b2[...],
                  seg_g[...], seg_be[...])
    feat = jnp.maximum(h + sg, 0.0)
    feat_out[...] = feat
    _write_y(feat, pre_w, suc_w, y_out)


def _tc_block_body(agg_ref, fprev_ref, ctr_w, n_g, n_b, c2_w, c2_b, c2_g, c2_be,
                   *rest):
    with_y = len(rest) == 4
    fprev = fprev_ref[...]
    temp = jnp.concatenate([agg_ref[0], agg_ref[1]], axis=-1)
    temp = temp + jnp.dot(fprev, ctr_w[...], preferred_element_type=jnp.float32)
    f1 = jnp.maximum(_gn_rows(temp, n_g[...], n_b[...]), 0.0)
    f2 = _gn_rows(jnp.dot(f1, c2_w[...], preferred_element_type=jnp.float32) + c2_b[...],
                  c2_g[...], c2_be[...])
    feat = jnp.maximum(f2 + fprev, 0.0)
    if with_y:
        pre_w, suc_w, feat_out, y_out = rest
        feat_out[...] = feat
        _write_y(feat, pre_w, suc_w, y_out)
    else:
        (feat_out,) = rest
        feat_out[...] = feat


def _full(shp):
    return pl.BlockSpec(shp, lambda r: (0,) * len(shp))


_ROWB = pl.BlockSpec((ROWS, D), lambda r: (r, 0))
_YB = pl.BlockSpec((2, NSETS, ROWS, CH), lambda r: (0, 0, r, 0))
_AGGB = pl.BlockSpec((2, ROWS, CH), lambda r: (0, r, 0))
_PAR = [_full((1, D))] * 2
_GRID = (NP // ROWS,)

_Y_SHAPE = jax.ShapeDtypeStruct((2, NSETS, NP, CH), jnp.float32)

_tc_first = pl.pallas_call(
    _tc_first_body,
    grid=_GRID,
    in_specs=[
        pl.BlockSpec((ROWS, 2), lambda r: (r, 0)),
        pl.BlockSpec((ROWS, 2), lambda r: (r, 0)),
        _full((2, D)), _full((1, D)), _full((D, D)), _full((1, D)), _full((1, D)), _full((1, D)),
        _full((2, D)), _full((1, D)), _full((D, D)), _full((1, D)), _full((1, D)), _full((1, D)),
        _full((S, D, D)), _full((S, D, D)),
    ],
    out_specs=[_ROWB, _YB],
    out_shape=[jax.ShapeDtypeStruct((NP, D), jnp.float32), _Y_SHAPE],
)

_MID_IN = [_AGGB, _ROWB, _full((D, D))] + _PAR + [_full((D, D))] + [_full((1, D))] * 3

_tc_mid = pl.pallas_call(
    _tc_block_body,
    grid=_GRID,
    in_specs=_MID_IN + [_full((S, D, D)), _full((S, D, D))],
    out_specs=[_ROWB, _YB],
    out_shape=[jax.ShapeDtypeStruct((NP, D), jnp.float32), _Y_SHAPE],
)

_tc_last = pl.pallas_call(
    _tc_block_body,
    grid=_GRID,
    in_specs=_MID_IN,
    out_specs=[_ROWB],
    out_shape=[jax.ShapeDtypeStruct((NP, D), jnp.float32)],
)


@functools.cache
def _make_sc_aggregate():
    return functools.partial(
        pl.kernel,
        out_type=jax.ShapeDtypeStruct((2, NP, CH), jnp.float32),
        mesh=plsc.VectorSubcoreMesh(core_axis_name="c", subcore_axis_name="s"),
        scratch_types=[
            pltpu.VMEM((NCHUNK, ECHUNK), jnp.int32),
            pltpu.VMEM((NCHUNK, ECHUNK), jnp.int32),
            pltpu.VMEM((ECHUNK, CH), jnp.float32),
        ] + [pltpu.VMEM((ECHUNK, CH), jnp.float32)] * NBUF
          + [pltpu.VMEM_SHARED((AROWS, CH), jnp.float32)]
          + [pltpu.SemaphoreType.DMA] * (2 * NBUF),
    )(_sc_aggregate_body)


def _sc_aggregate_body(y_hbm, u_hbm, v_hbm, out_hbm, u_vm, v_vm, zbuf, *rest):
    bufs = rest[:NBUF]
    accum = rest[NBUF]
    sem_g = rest[NBUF + 1:NBUF + 1 + NBUF]
    sem_s = rest[NBUF + 1 + NBUF:]
    c = lax.axis_index("c")
    s = lax.axis_index("s")

    pltpu.sync_copy(v_hbm.at[s], v_vm)

    def zrow(r, carry):
        for kk in range(CH // 16):
            zbuf[r, pl.ds(kk * 16, 16)] = jnp.zeros((16,), jnp.float32)
        return carry
    lax.fori_loop(0, ECHUNK, zrow, 0)

    def gather(t, b):
        return pltpu.async_copy(y_hbm.at[c].at[v_vm.at[t]], bufs[b], sem_g[b])

    def gather_wait(t, b):
        pltpu.make_async_copy(y_hbm.at[c].at[v_vm.at[t]], bufs[b],
                              sem_g[b]).wait()

    def scatter(t, b):
        return pltpu.async_copy(bufs[b], accum.at[u_vm.at[t]], sem_s[b],
                                add=True)

    for rp in range(2):
        pltpu.sync_copy(u_hbm.at[rp, s], u_vm)

        base = s * ZPT
        pltpu.sync_copy(zbuf, accum.at[pl.ds(base, ECHUNK)])
        pltpu.sync_copy(zbuf, accum.at[pl.ds(base + ECHUNK, ECHUNK)])
        pltpu.sync_copy(zbuf.at[pl.ds(0, ZPT - 2 * ECHUNK)],
                        accum.at[pl.ds(base + 2 * ECHUNK, ZPT - 2 * ECHUNK)])
        plsc.subcore_barrier()

        for b in range(NBUF):
            gather(b, b)

        def ring(i, carry):
            t0 = i * NBUF
            scats = []
            for b in range(NBUF):
                gather_wait(t0 + b, b)
                scats.append(scatter(t0 + b, b))
            for b in range(NBUF):
                scats[b].wait()
                gather(t0 + NBUF + b, b)
            return carry
        lax.fori_loop(0, NCHUNK // NBUF - 1, ring, 0)

        t0 = NCHUNK - NBUF
        scats = []
        for b in range(NBUF):
            gather_wait(t0 + b, b)
            scats.append(scatter(t0 + b, b))
        for b in range(NBUF):
            scats[b].wait()
        plsc.subcore_barrier()

        wbase = s * WPT
        for off, ln in ((0, ECHUNK), (ECHUNK, ECHUNK),
                        (2 * ECHUNK, WPT - 2 * ECHUNK)):
            b = (off // ECHUNK) % NBUF
            pltpu.sync_copy(accum.at[pl.ds(wbase + off, ln)],
                            bufs[b].at[pl.ds(0, ln)])
            pltpu.sync_copy(bufs[b].at[pl.ds(0, ln)],
                            out_hbm.at[c, pl.ds(rp * HALF + wbase + off, ln)])
        plsc.subcore_barrier()


def kernel(ctrs, feats, pre_u, pre_v, suc_u, suc_v, idcs,
           in_W1, in_b1, in_W2, in_b2, in_g, in_be,
           seg_W1, seg_b1, seg_W2, seg_b2, seg_g, seg_be,
           ctr_W, pre_W, suc_W, norm_g, norm_b,
           ctr2_W, ctr2_b, ctr2_g, ctr2_be):
    f32 = jnp.float32
    ctrs_p = jnp.pad(ctrs.astype(f32), ((0, NP - N), (0, 0)))
    feats_p = jnp.pad(feats.astype(f32), ((0, NP - N), (0, 0)))
    u_all = jnp.concatenate([pre_u, suc_u], axis=0).astype(jnp.int32)
    v_all = jnp.concatenate([pre_v, suc_v], axis=0).astype(jnp.int32)
    u_pad = jnp.pad(u_all, ((0, 0), (0, EP - N)), constant_values=NP - 1)
    v_pad = jnp.pad(v_all, ((0, 0), (0, EP - N)))
    v_flat = v_pad + (jnp.arange(NSETS, dtype=jnp.int32) * NP)[:, None]
    trash = HALF + (jnp.arange(EP, dtype=jnp.int32) % TRROWS)[None, :]
    u_2p = jnp.stack([
        jnp.where(u_pad < HALF, u_pad, trash),
        jnp.where(u_pad >= HALF, u_pad - HALF, trash),
    ])
    u_p = u_2p.reshape(2, NSETS, TILES, QCH, ECHUNK) \
              .transpose(0, 2, 1, 3, 4).reshape(2, TILES, NCHUNK, ECHUNK)
    v_p = v_flat.reshape(NSETS, TILES, QCH, ECHUNK) \
                .transpose(1, 0, 2, 3).reshape(TILES, NCHUNK, ECHUNK)

    def r2(x):
        return x.reshape(1, D).astype(f32)

    feat, y = _tc_first(ctrs_p, feats_p,
                        in_W1, r2(in_b1), in_W2, r2(in_b2), r2(in_g), r2(in_be),
                        seg_W1, r2(seg_b1), seg_W2, r2(seg_b2), r2(seg_g), r2(seg_be),
                        pre_W[0], suc_W[0])
    sc_aggregate = _make_sc_aggregate()
    for i in range(4):
        agg = sc_aggregate(y.reshape(2, NSETS * NP, CH), u_p, v_p)
        args = (agg, feat, ctr_W[i],
                r2(norm_g[i]), r2(norm_b[i]),
                ctr2_W[i], r2(ctr2_b[i]), r2(ctr2_g[i]), r2(ctr2_be[i]))
        if i < 3:
            feat, y = _tc_mid(*args, pre_W[i + 1], suc_W[i + 1])
        else:
            (feat,) = _tc_last(*args)
    return feat[:N]


# speedup vs baseline: 7.1538x; 2.1284x over previous
import functools

import jax
import jax.numpy as jnp
from jax import lax
from jax.experimental import pallas as pl
from jax.experimental.pallas import tpu as pltpu
from jax.experimental.pallas import tpu_sc as plsc

N = 10000
S = 6
D = 256
NP = 10240
EP = 10240
NSETS = 2 * S
TILES = 16
ECHUNK = 128
QCH = EP // TILES // ECHUNK
NCHUNK = NSETS * QCH
CTOT = NSETS * EP // ECHUNK
NBUF = 3
CH = D // 2
HALF = NP // 2
TRROWS = 128
AROWS = HALF + TRROWS
ZPT = AROWS // TILES
WPT = HALF // TILES
ROWS = 640


def _gn_rows(x, g, b):
    m = jnp.mean(x, axis=-1, keepdims=True)
    v = jnp.mean((x - m) ** 2, axis=-1, keepdims=True)
    return (x - m) / jnp.sqrt(v + 1e-5) * g + b


def _write_y(feat, pre_w, suc_w, y_out):
    for k in range(S):
        z = jnp.dot(feat, pre_w[k], preferred_element_type=jnp.float32)
        y_out[0, k] = z[:, :CH]
        y_out[1, k] = z[:, CH:]
        z = jnp.dot(feat, suc_w[k], preferred_element_type=jnp.float32)
        y_out[0, S + k] = z[:, :CH]
        y_out[1, S + k] = z[:, CH:]


def _tc_first_body(ctrs_ref, feats_ref, in_w1, in_b1, in_w2, in_b2, in_g, in_be,
                   seg_w1, seg_b1, seg_w2, seg_b2, seg_g, seg_be,
                   pre_w, suc_w, feat_out, y_out):
    ctrs = ctrs_ref[...]
    feats = feats_ref[...]
    w1 = in_w1[...]
    h = jnp.maximum(ctrs[:, 0:1] * w1[0:1, :] + ctrs[:, 1:2] * w1[1:2, :] + in_b1[...], 0.0)
    h = _gn_rows(jnp.dot(h, in_w2[...], preferred_element_type=jnp.float32) + in_b2[...],
                 in_g[...], in_be[...])
    w1 = seg_w1[...]
    sg = jnp.maximum(feats[:, 0:1] * w1[0:1, :] + feats[:, 1:2] * w1[1:2, :] + seg_b1[...], 0.0)
    sg = _gn_rows(jnp.dot(sg, seg_w2[...], preferred_element_type=jnp.float32) + seg_b2[...],
                  seg_g[...], seg_be[...])
    feat = jnp.maximum(h + sg, 0.0)
    feat_out[...] = feat
    _write_y(feat, pre_w, suc_w, y_out)


def _tc_block_body(agg_ref, fprev_ref, ctr_w, n_g, n_b, c2_w, c2_b, c2_g, c2_be,
                   *rest):
    with_y = len(rest) == 4
    fprev = fprev_ref[...]
    temp = jnp.concatenate([agg_ref[0], agg_ref[1]], axis=-1)
    temp = temp + jnp.dot(fprev, ctr_w[...], preferred_element_type=jnp.float32)
    f1 = jnp.maximum(_gn_rows(temp, n_g[...], n_b[...]), 0.0)
    f2 = _gn_rows(jnp.dot(f1, c2_w[...], preferred_element_type=jnp.float32) + c2_b[...],
                  c2_g[...], c2_be[...])
    feat = jnp.maximum(f2 + fprev, 0.0)
    if with_y:
        pre_w, suc_w, feat_out, y_out = rest
        feat_out[...] = feat
        _write_y(feat, pre_w, suc_w, y_out)
    else:
        (feat_out,) = rest
        feat_out[...] = feat


def _full(shp):
    return pl.BlockSpec(shp, lambda r: (0,) * len(shp))


_ROWB = pl.BlockSpec((ROWS, D), lambda r: (r, 0))
_YB = pl.BlockSpec((2, NSETS, ROWS, CH), lambda r: (0, 0, r, 0))
_AGGB = pl.BlockSpec((2, ROWS, CH), lambda r: (0, r, 0))
_PAR = [_full((1, D))] * 2
_GRID = (NP // ROWS,)

_Y_SHAPE = jax.ShapeDtypeStruct((2, NSETS, NP, CH), jnp.float32)

_tc_first = pl.pallas_call(
    _tc_first_body,
    grid=_GRID,
    in_specs=[
        pl.BlockSpec((ROWS, 2), lambda r: (r, 0)),
        pl.BlockSpec((ROWS, 2), lambda r: (r, 0)),
        _full((2, D)), _full((1, D)), _full((D, D)), _full((1, D)), _full((1, D)), _full((1, D)),
        _full((2, D)), _full((1, D)), _full((D, D)), _full((1, D)), _full((1, D)), _full((1, D)),
        _full((S, D, D)), _full((S, D, D)),
    ],
    out_specs=[_ROWB, _YB],
    out_shape=[jax.ShapeDtypeStruct((NP, D), jnp.float32), _Y_SHAPE],
)

_MID_IN = [_AGGB, _ROWB, _full((D, D))] + _PAR + [_full((D, D))] + [_full((1, D))] * 3

_tc_mid = pl.pallas_call(
    _tc_block_body,
    grid=_GRID,
    in_specs=_MID_IN + [_full((S, D, D)), _full((S, D, D))],
    out_specs=[_ROWB, _YB],
    out_shape=[jax.ShapeDtypeStruct((NP, D), jnp.float32), _Y_SHAPE],
)

_tc_last = pl.pallas_call(
    _tc_block_body,
    grid=_GRID,
    in_specs=_MID_IN,
    out_specs=[_ROWB],
    out_shape=[jax.ShapeDtypeStruct((NP, D), jnp.float32)],
)


@functools.cache
def _make_sc_aggregate():
    return functools.partial(
        pl.kernel,
        out_type=jax.ShapeDtypeStruct((2, NP, CH), jnp.float32),
        mesh=plsc.VectorSubcoreMesh(core_axis_name="c", subcore_axis_name="s"),
        scratch_types=[
            pltpu.VMEM((NCHUNK, ECHUNK), jnp.int32),
            pltpu.VMEM((NCHUNK, ECHUNK), jnp.int32),
            pltpu.VMEM((2, 16), jnp.int32),
            pltpu.VMEM((ECHUNK, CH), jnp.float32),
        ] + [pltpu.VMEM((ECHUNK, CH), jnp.float32)] * NBUF
          + [pltpu.VMEM_SHARED((AROWS, CH), jnp.float32)]
          + [pltpu.SemaphoreType.DMA] * (2 * NBUF),
    )(_sc_aggregate_body)


def _sc_aggregate_body(y_hbm, u_hbm, v_hbm, lim_hbm, out_hbm,
                       u_vm, v_vm, lim_vm, zbuf, *rest):
    bufs = rest[:NBUF]
    accum = rest[NBUF]
    sem_g = rest[NBUF + 1:NBUF + 1 + NBUF]
    sem_s = rest[NBUF + 1 + NBUF:]
    c = lax.axis_index("c")
    s = lax.axis_index("s")

    pltpu.sync_copy(v_hbm.at[s], v_vm)
    pltpu.sync_copy(lim_hbm, lim_vm)
    c0 = lim_vm[0][0]
    cb = lim_vm[1][0]
    k0 = jnp.clip((c0 - s + TILES - 1) // TILES, 0, NCHUNK)
    k1 = jnp.clip((cb - s + TILES - 1) // TILES, 0, NCHUNK)

    def zrow(r, carry):
        for kk in range(CH // 16):
            zbuf[r, pl.ds(kk * 16, 16)] = jnp.zeros((16,), jnp.float32)
        return carry
    lax.fori_loop(0, ECHUNK, zrow, 0)

    def gather(t, b):
        return pltpu.async_copy(y_hbm.at[c].at[v_vm.at[t]], bufs[b], sem_g[b])

    def gather_wait(t, b):
        pltpu.make_async_copy(y_hbm.at[c].at[v_vm.at[t]], bufs[b],
                              sem_g[b]).wait()

    def scatter(t, b):
        return pltpu.async_copy(bufs[b], accum.at[u_vm.at[t]], sem_s[b],
                                add=True)

    def scatter_wait(t, b):
        pltpu.make_async_copy(bufs[b], accum.at[u_vm.at[t]], sem_s[b]).wait()

    def run_pass(rp, kstart, cnt):
        pltpu.sync_copy(u_hbm.at[rp, s], u_vm)

        base = s * ZPT
        pltpu.sync_copy(zbuf, accum.at[pl.ds(base, ECHUNK)])
        pltpu.sync_copy(zbuf, accum.at[pl.ds(base + ECHUNK, ECHUNK)])
        pltpu.sync_copy(zbuf.at[pl.ds(0, ZPT - 2 * ECHUNK)],
                        accum.at[pl.ds(base + 2 * ECHUNK, ZPT - 2 * ECHUNK)])
        plsc.subcore_barrier()

        for b in range(NBUF):
            @pl.when(b < cnt)
            def _():
                gather(kstart + b, b)

        def ring(i, carry):
            t0 = i * NBUF
            for b in range(NBUF):
                k = t0 + b

                @pl.when(k < cnt)
                def _():
                    gather_wait(kstart + k, b)
                    scatter(kstart + k, b)
            for b in range(NBUF):
                k = t0 + b

                @pl.when(k < cnt)
                def _():
                    scatter_wait(kstart + k, b)
                kn = t0 + NBUF + b

                @pl.when(kn < cnt)
                def _():
                    gather(kstart + kn, b)
            return carry
        lax.fori_loop(0, (NCHUNK + NBUF - 1) // NBUF + 1, ring, 0)
        plsc.subcore_barrier()

        wbase = s * WPT
        for off, ln in ((0, ECHUNK), (ECHUNK, ECHUNK),
                        (2 * ECHUNK, WPT - 2 * ECHUNK)):
            b = (off // ECHUNK) % NBUF
            pltpu.sync_copy(accum.at[pl.ds(wbase + off, ln)],
                            bufs[b].at[pl.ds(0, ln)])
            pltpu.sync_copy(bufs[b].at[pl.ds(0, ln)],
                            out_hbm.at[c, pl.ds(rp * HALF + wbase + off, ln)])
        plsc.subcore_barrier()

    run_pass(0, 0, k0)
    run_pass(1, k1, NCHUNK - k1)


def kernel(ctrs, feats, pre_u, pre_v, suc_u, suc_v, idcs,
           in_W1, in_b1, in_W2, in_b2, in_g, in_be,
           seg_W1, seg_b1, seg_W2, seg_b2, seg_g, seg_be,
           ctr_W, pre_W, suc_W, norm_g, norm_b,
           ctr2_W, ctr2_b, ctr2_g, ctr2_be):
    f32 = jnp.float32
    ctrs_p = jnp.pad(ctrs.astype(f32), ((0, NP - N), (0, 0)))
    feats_p = jnp.pad(feats.astype(f32), ((0, NP - N), (0, 0)))
    u_all = jnp.concatenate([pre_u, suc_u], axis=0).astype(jnp.int32)
    v_all = jnp.concatenate([pre_v, suc_v], axis=0).astype(jnp.int32)
    u_pad = jnp.pad(u_all, ((0, 0), (0, EP - N)), constant_values=NP - 1)
    v_pad = jnp.pad(v_all, ((0, 0), (0, EP - N)))
    v_flat = (v_pad + (jnp.arange(NSETS, dtype=jnp.int32) * NP)[:, None]).reshape(-1)
    u_flat = u_pad.reshape(-1)
    key = (u_flat >= HALF).astype(jnp.int32)
    packed = (key << 30) | (v_flat << 13) | (u_flat - key * HALF)
    packed = jax.lax.sort(packed)
    key_s = packed >> 30
    v_s = (packed >> 13) & 0x1FFFF
    u_local = packed & 0x1FFF
    n0 = jnp.sum(1 - key)
    lim = jnp.stack([(n0 + ECHUNK - 1) // ECHUNK, n0 // ECHUNK]).astype(jnp.int32)
    lim16 = jnp.broadcast_to(lim[:, None], (2, 16))
    trash = HALF + (jnp.arange(NSETS * EP, dtype=jnp.int32) % TRROWS)
    u_2p = jnp.stack([
        jnp.where(key_s == 0, u_local, trash),
        jnp.where(key_s == 1, u_local, trash),
    ])
    u_p = u_2p.reshape(2, NCHUNK, TILES, ECHUNK).transpose(0, 2, 1, 3)
    v_p = v_s.reshape(NCHUNK, TILES, ECHUNK).transpose(1, 0, 2)

    def r2(x):
        return x.reshape(1, D).astype(f32)

    feat, y = _tc_first(ctrs_p, feats_p,
                        in_W1, r2(in_b1), in_W2, r2(in_b2), r2(in_g), r2(in_be),
                        seg_W1, r2(seg_b1), seg_W2, r2(seg_b2), r2(seg_g), r2(seg_be),
                        pre_W[0], suc_W[0])
    sc_aggregate = _make_sc_aggregate()
    for i in range(4):
        agg = sc_aggregate(y.reshape(2, NSETS * NP, CH), u_p, v_p, lim16)
        args = (agg, feat, ctr_W[i],
                r2(norm_g[i]), r2(norm_b[i]),
                ctr2_W[i], r2(ctr2_b[i]), r2(ctr2_g[i]), r2(ctr2_be[i]))
        if i < 3:
            feat, y = _tc_mid(*args, pre_W[i + 1], suc_W[i + 1])
        else:
            (feat,) = _tc_last(*args)
    return feat[:N]
